# baseline (speedup 1.0000x reference)
import functools

import jax
import jax.numpy as jnp
from jax import lax
from jax.experimental import pallas as pl
from jax.experimental.pallas import tpu as pltpu
from jax.experimental.pallas import tpu_sc as plsc

N_SRC0 = 10000
N_DST0 = 4000
N_DST1 = 2000
HID = 64
HEADS = 4
OUT = 128
D = HID * HEADS

NC = 2
NS = 16
NW = NC * NS
B = 64

f32 = jnp.float32
i32 = jnp.int32


def _tc1_body(img_ref, txt_ref, wimgT_ref, wblkT_ref, wv_ref, bv_ref, we_ref,
              be_ref, wfc1T_ref, a1_ref, z1_ref, s_ref, smax_ref):
    i = pl.program_id(0)
    fi = jnp.dot(img_ref[...], wimgT_ref[...], preferred_element_type=f32)
    ti = jnp.dot(txt_ref[...], wblkT_ref[...], preferred_element_type=f32)
    av = jax.nn.sigmoid(jnp.dot(fi, wv_ref[...], preferred_element_type=f32)
                        + bv_ref[...])
    ae = jax.nn.sigmoid(jnp.dot(ti, we_ref[...], preferred_element_type=f32)
                        + be_ref[...])
    fused = av * fi + ae * ti
    z1 = jnp.dot(fused, wfc1T_ref[...], preferred_element_type=f32)
    s = jnp.dot(z1, a1_ref[...], preferred_element_type=f32)
    z1_ref[...] = z1
    s_ref[...] = s
    cm = jnp.max(s, axis=0, keepdims=True)
    col = lax.broadcasted_iota(i32, (1, 8), 1)
    cm = jnp.where((col < 4) | (i < N_DST0 // 400), cm, -3.0e38)

    @pl.when(i == 0)
    def _():
        smax_ref[...] = cm

    @pl.when(i > 0)
    def _():
        smax_ref[...] = jnp.maximum(smax_ref[...], cm)


def _tc1(img, txt, wimgT, wblkT, wv, bv2, we, be2, wfc1T, a1):
    R = 400
    grid = (N_SRC0 // R,)
    full = lambda shape: pl.BlockSpec(shape, lambda i: (0, 0))
    return pl.pallas_call(
        _tc1_body,
        grid=grid,
        in_specs=[
            pl.BlockSpec((R, 256), lambda i: (i, 0)),
            pl.BlockSpec((R, 256), lambda i: (i, 0)),
            full((256, 256)), full((256, 256)), full((256, 256)),
            full((1, 256)), full((256, 256)), full((1, 256)),
            full((256, 256)), full((256, 8)),
        ],
        out_specs=[
            pl.BlockSpec((R, 256), lambda i: (i, 0)),
            pl.BlockSpec((R, 8), lambda i: (i, 0)),
            pl.BlockSpec((1, 8), lambda i: (0, 0)),
        ],
        out_shape=[
            jax.ShapeDtypeStruct((N_SRC0, 256), f32),
            jax.ShapeDtypeStruct((N_SRC0, 8), f32),
            jax.ShapeDtypeStruct((1, 8), f32),
        ],
    )(img, txt, wimgT, wblkT, wv, bv2, we, be2, wfc1T, a1)


def _make_sc_edge(n_src, dz, hid, heads, n_dst_pad, nb_per):
    rows_pt = n_dst_pad // NS
    n_chunk = dz // 16
    mesh = plsc.VectorSubcoreMesh(core_axis_name="c", subcore_axis_name="s")

    @functools.partial(
        pl.kernel,
        out_type=[
            jax.ShapeDtypeStruct((NC, n_dst_pad, dz), f32),
            jax.ShapeDtypeStruct((NC, n_dst_pad, 16), f32),
        ],
        mesh=mesh,
        scratch_types=[
            pltpu.VMEM((2, B, 16), f32),
            pltpu.VMEM((2, B, 16), f32),
            pltpu.VMEM((nb_per + 2, B), i32),
            pltpu.VMEM((nb_per + 2, B), i32),
            pltpu.VMEM((2, B, dz), f32),
            pltpu.VMEM((2, B, 16), f32),
            pltpu.VMEM((16,), f32),
            pltpu.SemaphoreType.DMA,
            pltpu.SemaphoreType.DMA,
            pltpu.VMEM_SHARED((n_dst_pad, dz), f32),
            pltpu.VMEM_SHARED((n_dst_pad, 16), f32),
        ],
        compiler_params=pltpu.CompilerParams(needs_layout_passes=False,
                                             use_tc_tiling_on_sc=False),
    )
    def sc_edge(z_hbm, stab_hbm, m_hbm, sidx_hbm, didx_hbm, zrows_hbm,
                zden_hbm, num_out, den_out, srows_v, drows_v, sidx_v,
                didx_v, rows_v, wexp_v, m_v, sem0, sem1, acc_sh, den_sh):
        c = lax.axis_index("c")
        s = lax.axis_index("s")
        w = s * NC + c
        sems = (sem0, sem1)

        pltpu.sync_copy(zrows_hbm, acc_sh.at[pl.ds(s * rows_pt, rows_pt)])
        pltpu.sync_copy(zden_hbm, den_sh.at[pl.ds(s * rows_pt, rows_pt)])
        pltpu.sync_copy(zden_hbm.at[pl.ds(0, B)], wexp_v.at[0])
        pltpu.sync_copy(zden_hbm.at[pl.ds(0, B)], wexp_v.at[1])

        pltpu.sync_copy(m_hbm, m_v)
        pltpu.sync_copy(sidx_hbm.at[w], sidx_v)
        pltpu.sync_copy(didx_hbm.at[w], didx_v)
        plsc.subcore_barrier()

        lane = lax.iota(i32, 16)
        m_vec = m_v[...]
        ms = [m_vec[h] for h in range(heads)]

        def gathers(p, jb):
            pltpu.async_copy(stab_hbm.at[sidx_v.at[jb]], srows_v.at[p],
                             sems[p])
            pltpu.async_copy(stab_hbm.at[didx_v.at[jb]], drows_v.at[p],
                             sems[p])
            pltpu.async_copy(z_hbm.at[sidx_v.at[jb]], rows_v.at[p], sems[p])

        def wait_gathers(p, jb):
            pltpu.make_async_copy(stab_hbm.at[sidx_v.at[jb]], srows_v.at[p],
                                  sems[p]).wait()
            pltpu.make_async_copy(stab_hbm.at[didx_v.at[jb]], drows_v.at[p],
                                  sems[p]).wait()
            pltpu.make_async_copy(z_hbm.at[sidx_v.at[jb]], rows_v.at[p],
                                  sems[p]).wait()

        def process(p, jb):
            srows_p, drows_p = srows_v.at[p], drows_v.at[p]
            rows_p, wexp_p = rows_v.at[p], wexp_v.at[p]
            for k in range(B // 16):
                ev = lane + k * 16
                for h in range(heads):
                    ss = plsc.load_gather(srows_p, [ev, lane * 0 + h])
                    sd = plsc.load_gather(drows_p, [ev, lane * 0 + 4 + h])
                    x = ss + sd
                    e = jnp.where(x > 0, x, x * 0.01)
                    wv = jnp.exp(e - ms[h])
                    plsc.store_scatter(wexp_p, [ev, lane * 0 + h], wv)

            @plsc.parallel_loop(0, B, 1, unroll=4)
            def scale(e):
                wrow = wexp_p[e, pl.ds(0, 16)]
                wh = [wrow[h] for h in range(heads)]
                for cdx in range(n_chunk):
                    h = (cdx * 16) // hid
                    rows_p[e, pl.ds(cdx * 16, 16)] = (
                        rows_p[e, pl.ds(cdx * 16, 16)] * wh[h])

            pltpu.sync_copy(rows_p, acc_sh.at[didx_v.at[jb]], add=True)
            pltpu.sync_copy(wexp_p, den_sh.at[didx_v.at[jb]], add=True)

        gathers(0, 0)
        gathers(1, 1)

        def pipelined(j2, carry):
            jb0 = 2 * j2
            wait_gathers(0, jb0)
            process(0, jb0)
            gathers(0, jb0 + 2)
            wait_gathers(1, jb0 + 1)
            process(1, jb0 + 1)
            gathers(1, jb0 + 3)
            return carry

        lax.fori_loop(0, nb_per // 2, pipelined, 0)
        wait_gathers(0, nb_per)
        wait_gathers(1, nb_per + 1)
        plsc.subcore_barrier()

        pltpu.sync_copy(acc_sh.at[pl.ds(s * rows_pt, rows_pt)],
                        num_out.at[c, pl.ds(s * rows_pt, rows_pt)])
        pltpu.sync_copy(den_sh.at[pl.ds(s * rows_pt, rows_pt)],
                        den_out.at[c, pl.ds(s * rows_pt, rows_pt)])

    return sc_edge


def _tc2_body(num_ref, den_ref, r1_ref, wfc2T_ref, a2_ref, z2_ref, s2_ref,
              smax2_ref):
    i = pl.program_id(0)
    num = num_ref[0] + num_ref[1]
    den16 = den_ref[0] + den_ref[1]
    denf = jnp.dot(den16, r1_ref[...], preferred_element_type=f32)
    denf = jnp.where(denf == 0.0, 1.0, denf)
    h1 = num / denf
    h1 = jnp.where(h1 > 0, h1, jnp.exp(jnp.minimum(h1, 0.0)) - 1.0)
    z2 = jnp.dot(h1, wfc2T_ref[...], preferred_element_type=f32)
    s2 = jnp.dot(z2, a2_ref[...], preferred_element_type=f32)
    z2_ref[...] = z2
    s2_ref[...] = s2
    cm = jnp.max(s2, axis=0, keepdims=True)
    col = lax.broadcasted_iota(i32, (1, 8), 1)
    cm = jnp.where((col < 4) | (i < N_DST1 // 400), cm, -3.0e38)

    @pl.when(i == 0)
    def _():
        smax2_ref[...] = cm

    @pl.when(i > 0)
    def _():
        smax2_ref[...] = jnp.maximum(smax2_ref[...], cm)


def _tc2(num1, den1, r1, wfc2T, a2):
    R = 400
    full = lambda shape: pl.BlockSpec(shape, lambda i: tuple(0 for _ in shape))
    return pl.pallas_call(
        _tc2_body,
        grid=(N_DST0 // R,),
        in_specs=[
            pl.BlockSpec((2, R, 256), lambda i: (0, i, 0)),
            pl.BlockSpec((2, R, 16), lambda i: (0, i, 0)),
            full((16, 256)), full((256, 128)), full((128, 8)),
        ],
        out_specs=[
            pl.BlockSpec((R, 128), lambda i: (i, 0)),
            pl.BlockSpec((R, 8), lambda i: (i, 0)),
            pl.BlockSpec((1, 8), lambda i: (0, 0)),
        ],
        out_shape=[
            jax.ShapeDtypeStruct((N_DST0, 128), f32),
            jax.ShapeDtypeStruct((N_DST0, 8), f32),
            jax.ShapeDtypeStruct((1, 8), f32),
        ],
    )(num1, den1, r1, wfc2T, a2)


def _tc3_body(num_ref, den_ref, r2_ref, out_ref):
    num = num_ref[0] + num_ref[1]
    den16 = den_ref[0] + den_ref[1]
    denf = jnp.dot(den16, r2_ref[...], preferred_element_type=f32)
    denf = jnp.where(denf == 0.0, 1.0, denf)
    out_ref[...] = num / denf


def _tc3(num2, den2, r2):
    R = 400
    return pl.pallas_call(
        _tc3_body,
        grid=(N_DST1 // R,),
        in_specs=[
            pl.BlockSpec((2, R, 128), lambda i: (0, i, 0)),
            pl.BlockSpec((2, R, 16), lambda i: (0, i, 0)),
            pl.BlockSpec((16, 128), lambda i: (0, 0)),
        ],
        out_specs=pl.BlockSpec((R, 128), lambda i: (i, 0)),
        out_shape=jax.ShapeDtypeStruct((N_DST1, 128), f32),
    )(num2, den2, r2)


def _prep_edges(src, dst, e_total, nb_per, dummy_dst):
    e_pad = NW * nb_per * B
    src_p = jnp.concatenate([src, jnp.zeros((e_pad - e_total,), i32)])
    dst_p = jnp.concatenate(
        [dst, jnp.full((e_pad - e_total,), dummy_dst, i32)])
    src_p = src_p.reshape(NW, nb_per, B)
    dst_p = dst_p.reshape(NW, nb_per, B)
    extra_s = jnp.zeros((NW, 2, B), i32)
    extra_d = jnp.full((NW, 2, B), dummy_dst, i32)
    return (jnp.concatenate([src_p, extra_s], axis=1),
            jnp.concatenate([dst_p, extra_d], axis=1))


def _head_matrix(att, heads, hid):
    d = heads * hid
    rows = jnp.arange(d)
    hod = rows // hid
    a = jnp.zeros((d, 8), f32)
    a = a.at[rows, hod].set(att[:, :hid].reshape(-1))
    a = a.at[rows, 4 + hod].set(att[:, hid:].reshape(-1))
    return a


def _expand_matrix(heads, hid, dz):
    r = jnp.zeros((16, dz), f32)
    cols = jnp.arange(dz)
    r = r.at[cols // hid, cols].set(1.0)
    return r


_NB1 = 80
_NB2 = 32

_sc_edge1 = _make_sc_edge(N_SRC0, 256, HID, HEADS, 4096, _NB1)
_sc_edge2 = _make_sc_edge(N_DST0, 128, OUT, 1, 2048, _NB2)


def kernel(image_features, text_features, edge_src0, edge_dst0, edge_src1,
           edge_dst1, W_img, W_blk, Wv, bv, We, be, Wfc1, Wattn1, Wfc2,
           Wattn2):
    a1 = _head_matrix(Wattn1, HEADS, HID)
    a2 = _head_matrix(Wattn2[None, :], 1, OUT)
    r1 = _expand_matrix(HEADS, HID, 256)
    r2 = _expand_matrix(1, OUT, 128)

    z1, s1, smax1 = _tc1(image_features, text_features, W_img.T, W_blk.T, Wv,
                         bv[None, :], We, be[None, :],
                         Wfc1.reshape(D, D).T, a1)

    m4 = jnp.maximum(smax1[0, :4] + smax1[0, 4:8], 0.0)
    m16 = jnp.zeros((16,), f32).at[:4].set(m4)
    sidx1, didx1 = _prep_edges(edge_src0, edge_dst0, 160000, _NB1, N_DST0)
    stab1 = jnp.pad(s1, ((0, 0), (0, 8)))
    num1, den1 = _sc_edge1(z1, stab1, m16, sidx1, didx1,
                           jnp.zeros((4096 // NS, 256), f32),
                           jnp.zeros((4096 // NS, 16), f32))

    z2, s2, smax2 = _tc2(num1, den1, r1, Wfc2.T, a2)

    m16b = jnp.zeros((16,), f32).at[0].set(
        jnp.maximum(smax2[0, 0] + smax2[0, 4], 0.0))
    sidx2, didx2 = _prep_edges(edge_src1, edge_dst1, 64000, _NB2, N_DST1)
    stab2 = jnp.pad(s2, ((0, 0), (0, 8)))
    num2, den2 = _sc_edge2(z2, stab2, m16b, sidx2, didx2,
                           jnp.zeros((2048 // NS, 128), f32),
                           jnp.zeros((2048 // NS, 16), f32))

    return _tc3(num2, den2, r2)


# speedup vs baseline: 1.4484x; 1.4484x over previous
import functools

import jax
import jax.numpy as jnp
from jax import lax
from jax.experimental import pallas as pl
from jax.experimental.pallas import tpu as pltpu
from jax.experimental.pallas import tpu_sc as plsc

N_SRC0 = 10000
N_DST0 = 4000
N_DST1 = 2000
HID = 64
HEADS = 4
OUT = 128
D = HID * HEADS

NC = 2
NS = 16
NW = NC * NS
B = 128

f32 = jnp.float32
i32 = jnp.int32


def _tc1_body(img_ref, txt_ref, wimgT_ref, wblkT_ref, wv_ref, bv_ref, we_ref,
              be_ref, wfc1T_ref, a1_ref, z1_ref, s_ref, smax_ref):
    i = pl.program_id(0)
    fi = jnp.dot(img_ref[...], wimgT_ref[...], preferred_element_type=f32)
    ti = jnp.dot(txt_ref[...], wblkT_ref[...], preferred_element_type=f32)
    av = jax.nn.sigmoid(jnp.dot(fi, wv_ref[...], preferred_element_type=f32)
                        + bv_ref[...])
    ae = jax.nn.sigmoid(jnp.dot(ti, we_ref[...], preferred_element_type=f32)
                        + be_ref[...])
    fused = av * fi + ae * ti
    z1 = jnp.dot(fused, wfc1T_ref[...], preferred_element_type=f32)
    s = jnp.dot(z1, a1_ref[...], preferred_element_type=f32)
    z1_ref[...] = z1.astype(jnp.bfloat16)
    s_ref[...] = s
    cm = jnp.max(s, axis=0, keepdims=True)
    col = lax.broadcasted_iota(i32, (1, 8), 1)
    cm = jnp.where((col < 4) | (i < N_DST0 // 400), cm, -3.0e38)

    @pl.when(i == 0)
    def _():
        smax_ref[...] = cm

    @pl.when(i > 0)
    def _():
        smax_ref[...] = jnp.maximum(smax_ref[...], cm)


def _tc1(img, txt, wimgT, wblkT, wv, bv2, we, be2, wfc1T, a1):
    R = 400
    grid = (N_SRC0 // R,)
    full = lambda shape: pl.BlockSpec(shape, lambda i: (0, 0))
    return pl.pallas_call(
        _tc1_body,
        grid=grid,
        in_specs=[
            pl.BlockSpec((R, 256), lambda i: (i, 0)),
            pl.BlockSpec((R, 256), lambda i: (i, 0)),
            full((256, 256)), full((256, 256)), full((256, 256)),
            full((1, 256)), full((256, 256)), full((1, 256)),
            full((256, 256)), full((256, 8)),
        ],
        out_specs=[
            pl.BlockSpec((R, 256), lambda i: (i, 0)),
            pl.BlockSpec((R, 8), lambda i: (i, 0)),
            pl.BlockSpec((1, 8), lambda i: (0, 0)),
        ],
        out_shape=[
            jax.ShapeDtypeStruct((N_SRC0, 256), jnp.bfloat16),
            jax.ShapeDtypeStruct((N_SRC0, 8), f32),
            jax.ShapeDtypeStruct((1, 8), f32),
        ],
    )(img, txt, wimgT, wblkT, wv, bv2, we, be2, wfc1T, a1)


def _make_sc_edge(n_src, dz, hid, heads, n_dst_pad, nb_per):
    rows_pt = n_dst_pad // NS
    waug = dz // 2 + 8
    mesh = plsc.VectorSubcoreMesh(core_axis_name="c", subcore_axis_name="s")

    @functools.partial(
        pl.kernel,
        out_type=[
            jax.ShapeDtypeStruct((NC, n_dst_pad, dz), f32),
            jax.ShapeDtypeStruct((NC, n_dst_pad, 16), f32),
        ],
        mesh=mesh,
        scratch_types=[
            pltpu.VMEM((2, B), i32),
            pltpu.VMEM((B, 16), f32),
            pltpu.VMEM((B, waug), i32),
            pltpu.VMEM((B, dz), f32),
            pltpu.VMEM((B, 16), f32),
            pltpu.VMEM((16,), f32),
            pltpu.VMEM_SHARED((n_dst_pad, dz), f32),
            pltpu.VMEM_SHARED((n_dst_pad, 16), f32),
        ],
        compiler_params=pltpu.CompilerParams(needs_layout_passes=False,
                                             use_tc_tiling_on_sc=False),
    )
    def sc_edge(z_hbm, stab_hbm, m_hbm, eidx_hbm, zrows_hbm, zden_hbm,
                num_out, den_out, eidx_v, drows_v, rows_v, stage_v, wexp_v,
                m_v, acc_sh, den_sh):
        c = lax.axis_index("c")
        s = lax.axis_index("s")
        w = s * NC + c

        pltpu.sync_copy(zrows_hbm, acc_sh.at[pl.ds(s * rows_pt, rows_pt)])
        pltpu.sync_copy(zden_hbm, den_sh.at[pl.ds(s * rows_pt, rows_pt)])
        pltpu.sync_copy(zden_hbm.at[pl.ds(0, B)], wexp_v)
        pltpu.sync_copy(m_hbm, m_v)
        plsc.subcore_barrier()

        lane = lax.iota(i32, 16)
        m_vec = m_v[...]
        ms = [m_vec[h] for h in range(heads)]
        mask_hi = jnp.full((16,), -65536, i32)

        def batch(jb, carry):
            pltpu.sync_copy(eidx_hbm.at[w, jb], eidx_v)
            pltpu.sync_copy(z_hbm.at[eidx_v.at[0]], rows_v)
            pltpu.sync_copy(stab_hbm.at[eidx_v.at[1]], drows_v)
            for k in range(B // 16):
                ev = lane + k * 16
                for h in range(heads):
                    ss = plsc.bitcast(
                        plsc.load_gather(rows_v,
                                         [ev, lane * 0 + dz // 2 + h]), f32)
                    sd = plsc.load_gather(drows_v, [ev, lane * 0 + 4 + h])
                    x = ss + sd
                    e = jnp.where(x > 0, x, x * 0.01)
                    wv = jnp.exp(e - ms[h])
                    plsc.store_scatter(wexp_v, [ev, lane * 0 + h], wv)

            @plsc.parallel_loop(0, B, 1, unroll=4)
            def scale(e):
                wrow = wexp_v[e, pl.ds(0, 16)]
                wh = [wrow[h] for h in range(heads)]
                for c2 in range(dz // 32):
                    h = (c2 * 32) // hid
                    v = rows_v[e, pl.ds(c2 * 16, 16)]
                    evenf = plsc.bitcast(lax.shift_left(v, 16), f32)
                    oddf = plsc.bitcast(lax.bitwise_and(v, mask_hi), f32)
                    stage_v[e, pl.ds(c2 * 32, 16)] = evenf * wh[h]
                    stage_v[e, pl.ds(c2 * 32 + 16, 16)] = oddf * wh[h]

            pltpu.sync_copy(stage_v, acc_sh.at[eidx_v.at[1]], add=True)
            pltpu.sync_copy(wexp_v, den_sh.at[eidx_v.at[1]], add=True)
            return carry

        lax.fori_loop(0, nb_per, batch, 0)
        plsc.subcore_barrier()

        pltpu.sync_copy(acc_sh.at[pl.ds(s * rows_pt, rows_pt)],
                        num_out.at[c, pl.ds(s * rows_pt, rows_pt)])
        pltpu.sync_copy(den_sh.at[pl.ds(s * rows_pt, rows_pt)],
                        den_out.at[c, pl.ds(s * rows_pt, rows_pt)])

    return sc_edge


def _tc2_body(num_ref, den_ref, r1_ref, wfc2T_ref, a2_ref, z2_ref, s2_ref,
              smax2_ref):
    i = pl.program_id(0)
    num = num_ref[0] + num_ref[1]
    den16 = den_ref[0] + den_ref[1]
    denf = jnp.dot(den16, r1_ref[...], preferred_element_type=f32)
    denf = jnp.where(denf == 0.0, 1.0, denf)
    h1 = num / denf
    h1 = jnp.where(h1 > 0, h1, jnp.exp(jnp.minimum(h1, 0.0)) - 1.0)
    z2 = jnp.dot(h1, wfc2T_ref[...], preferred_element_type=f32)
    s2 = jnp.dot(z2, a2_ref[...], preferred_element_type=f32)
    z2_ref[...] = z2.astype(jnp.bfloat16)
    s2_ref[...] = s2
    cm = jnp.max(s2, axis=0, keepdims=True)
    col = lax.broadcasted_iota(i32, (1, 8), 1)
    cm = jnp.where((col < 4) | (i < N_DST1 // 400), cm, -3.0e38)

    @pl.when(i == 0)
    def _():
        smax2_ref[...] = cm

    @pl.when(i > 0)
    def _():
        smax2_ref[...] = jnp.maximum(smax2_ref[...], cm)


def _tc2(num1, den1, r1, wfc2T, a2):
    R = 400
    full = lambda shape: pl.BlockSpec(shape, lambda i: tuple(0 for _ in shape))
    return pl.pallas_call(
        _tc2_body,
        grid=(N_DST0 // R,),
        in_specs=[
            pl.BlockSpec((2, R, 256), lambda i: (0, i, 0)),
            pl.BlockSpec((2, R, 16), lambda i: (0, i, 0)),
            full((16, 256)), full((256, 128)), full((128, 8)),
        ],
        out_specs=[
            pl.BlockSpec((R, 128), lambda i: (i, 0)),
            pl.BlockSpec((R, 8), lambda i: (i, 0)),
            pl.BlockSpec((1, 8), lambda i: (0, 0)),
        ],
        out_shape=[
            jax.ShapeDtypeStruct((N_DST0, 128), jnp.bfloat16),
            jax.ShapeDtypeStruct((N_DST0, 8), f32),
            jax.ShapeDtypeStruct((1, 8), f32),
        ],
    )(num1, den1, r1, wfc2T, a2)


def _tc3_body(num_ref, den_ref, r2_ref, punp_ref, out_ref):
    num = num_ref[0] + num_ref[1]
    den16 = den_ref[0] + den_ref[1]
    denf = jnp.dot(den16, r2_ref[...], preferred_element_type=f32)
    denf = jnp.where(denf == 0.0, 1.0, denf)
    out_ref[...] = jnp.dot(num / denf, punp_ref[...],
                           preferred_element_type=f32)


def _tc3(num2, den2, r2, punp):
    R = 400
    return pl.pallas_call(
        _tc3_body,
        grid=(N_DST1 // R,),
        in_specs=[
            pl.BlockSpec((2, R, 128), lambda i: (0, i, 0)),
            pl.BlockSpec((2, R, 16), lambda i: (0, i, 0)),
            pl.BlockSpec((16, 128), lambda i: (0, 0)),
            pl.BlockSpec((128, 128), lambda i: (0, 0)),
        ],
        out_specs=pl.BlockSpec((R, 128), lambda i: (i, 0)),
        out_shape=jax.ShapeDtypeStruct((N_DST1, 128), f32),
    )(num2, den2, r2, punp)


def _prep_edges(src, dst, e_total, nb_per, dummy_dst, n_garbage, n_src):
    e_pad = NW * nb_per * B
    pad_n = e_pad - e_total
    pad_i = jnp.arange(pad_n, dtype=i32)
    src_p = jnp.concatenate([src, pad_i % n_src])
    dst_p = jnp.concatenate([dst, dummy_dst + (pad_i % n_garbage)])
    return jnp.stack([src_p.reshape(NW, nb_per, B),
                      dst_p.reshape(NW, nb_per, B)], axis=2)


def _head_matrix(att, heads, hid):
    d = heads * hid
    rows = jnp.arange(d)
    hod = rows // hid
    a = jnp.zeros((d, 8), f32)
    a = a.at[rows, hod].set(att[:, :hid].reshape(-1))
    a = a.at[rows, 4 + hod].set(att[:, hid:].reshape(-1))
    return a


def _evenodd_perm(dz):
    j = jnp.arange(dz)
    base = (j // 32) * 32
    within = j % 32
    return base + jnp.where(within < 16, 2 * within, 2 * (within - 16) + 1)


def _expand_matrix(heads, hid, dz):
    r = jnp.zeros((16, dz), f32)
    cols = jnp.arange(dz)
    r = r.at[cols // hid, cols].set(1.0)
    return r


_NB1 = 40
_NB2 = 16

_sc_edge1 = _make_sc_edge(N_SRC0, 256, HID, HEADS, 4096, _NB1)
_sc_edge2 = _make_sc_edge(N_DST0, 128, OUT, 1, 2048, _NB2)


def kernel(image_features, text_features, edge_src0, edge_dst0, edge_src1,
           edge_dst1, W_img, W_blk, Wv, bv, We, be, Wfc1, Wattn1, Wfc2,
           Wattn2):
    a1 = _head_matrix(Wattn1, HEADS, HID)
    a2 = _head_matrix(Wattn2[None, :], 1, OUT)
    r1 = _expand_matrix(HEADS, HID, 256)
    r2 = _expand_matrix(1, OUT, 128)

    z1, s1, smax1 = _tc1(image_features, text_features, W_img.T, W_blk.T, Wv,
                         bv[None, :], We, be[None, :],
                         Wfc1.reshape(D, D).T, a1)

    m4 = jnp.maximum(smax1[0, :4] + smax1[0, 4:8], 0.0)
    m16 = jnp.zeros((16,), f32).at[:4].set(m4)
    eidx1 = _prep_edges(edge_src0, edge_dst0, 160000, _NB1, N_DST0, 96, N_SRC0)
    stab1 = jnp.pad(s1, ((0, 0), (0, 8)))
    z1aug = jnp.concatenate(
        [lax.bitcast_convert_type(z1.reshape(N_SRC0, 128, 2), i32),
         lax.bitcast_convert_type(s1, i32)], axis=1)
    num1, den1 = _sc_edge1(z1aug, stab1, m16, eidx1,
                           jnp.zeros((4096 // NS, 256), f32),
                           jnp.zeros((4096 // NS, 16), f32))

    wfc2T_perm = Wfc2.T[_evenodd_perm(256), :]
    z2, s2, smax2 = _tc2(num1, den1, r1, wfc2T_perm, a2)

    m16b = jnp.zeros((16,), f32).at[0].set(
        jnp.maximum(smax2[0, 0] + smax2[0, 4], 0.0))
    eidx2 = _prep_edges(edge_src1, edge_dst1, 64000, _NB2, N_DST1, 48, N_DST0)
    stab2 = jnp.pad(s2, ((0, 0), (0, 8)))
    z2aug = jnp.concatenate(
        [lax.bitcast_convert_type(z2.reshape(N_DST0, 64, 2), i32),
         lax.bitcast_convert_type(s2, i32)], axis=1)
    num2, den2 = _sc_edge2(z2aug, stab2, m16b, eidx2,
                           jnp.zeros((2048 // NS, 128), f32),
                           jnp.zeros((2048 // NS, 16), f32))

    p2 = _evenodd_perm(128)
    punp = jnp.zeros((128, 128), f32).at[jnp.arange(128), p2].set(1.0)
    return _tc3(num2, den2, r2, punp)
